# Initial kernel scaffold; baseline (speedup 1.0000x reference)
import jax
import jax.numpy as jnp
from jax.experimental import pallas as pl


def kernel(x, w_gate, w1, w2):
    raise NotImplementedError("write your pallas kernel here")


# speedup vs baseline: 1.0793x; 1.0793x over previous
import functools

import jax
import jax.numpy as jnp
from jax.experimental import pallas as pl
from jax.experimental.pallas import tpu as pltpu

D_MODEL = 1024
D_FF = 4096
N_EXP = 8
N_TOK = 4096

TB = 1024
FB = 1024


def _router_body(x_ref, wg_ref, gates_ref):
    x = x_ref[...]
    wg = wg_ref[...]
    logits = jnp.dot(x, wg, preferred_element_type=jnp.float32)
    m = jnp.max(logits, axis=-1, keepdims=True)
    e = jnp.exp(logits - m)
    s = jnp.sum(e, axis=-1, keepdims=True)
    p = e / s
    cols = jax.lax.broadcasted_iota(jnp.int32, p.shape, 1)
    v1 = jnp.max(p, axis=-1, keepdims=True)
    i1 = jnp.min(jnp.where(p == v1, cols, N_EXP), axis=-1, keepdims=True)
    p2 = jnp.where(cols == i1, -1.0, p)
    v2 = jnp.max(p2, axis=-1, keepdims=True)
    i2 = jnp.min(jnp.where(p2 == v2, cols, N_EXP), axis=-1, keepdims=True)
    denom = v1 + v2 + 1e-9
    gates = jnp.where(cols == i1, v1 / denom, 0.0) + jnp.where(cols == i2, v2 / denom, 0.0)
    gates_ref[...] = gates


def _moe_body(x_ref, gates_ref, w1_ref, w2_ref, out_ref):
    e = pl.program_id(1)
    f = pl.program_id(2)

    @pl.when((e == 0) & (f == 0))
    def _init():
        out_ref[...] = jnp.zeros_like(out_ref)

    x = x_ref[...]
    h = jnp.dot(x, w1_ref[0], preferred_element_type=jnp.float32)
    h = jnp.maximum(h, 0.0).astype(jnp.bfloat16)
    part = jnp.dot(h, w2_ref[0], preferred_element_type=jnp.float32)
    g_all = gates_ref[...]
    cols = jax.lax.broadcasted_iota(jnp.int32, g_all.shape, 1)
    gate = jnp.sum(jnp.where(cols == e, g_all, 0.0), axis=-1, keepdims=True)
    out_ref[...] += gate * part


@jax.jit
def kernel(x, w_gate, w1, w2):
    gates = pl.pallas_call(
        _router_body,
        grid=(N_TOK // TB,),
        in_specs=[
            pl.BlockSpec((TB, D_MODEL), lambda t: (t, 0)),
            pl.BlockSpec((D_MODEL, N_EXP), lambda t: (0, 0)),
        ],
        out_specs=pl.BlockSpec((TB, N_EXP), lambda t: (t, 0)),
        out_shape=jax.ShapeDtypeStruct((N_TOK, N_EXP), jnp.float32),
    )(x, w_gate)

    x_bf = x.astype(jnp.bfloat16)
    w1_bf = w1.astype(jnp.bfloat16)
    w2_bf = w2.astype(jnp.bfloat16)

    out = pl.pallas_call(
        _moe_body,
        grid=(N_TOK // TB, N_EXP, D_FF // FB),
        in_specs=[
            pl.BlockSpec((TB, D_MODEL), lambda t, e, f: (t, 0)),
            pl.BlockSpec((TB, N_EXP), lambda t, e, f: (t, 0)),
            pl.BlockSpec((1, D_MODEL, FB), lambda t, e, f: (e, 0, f)),
            pl.BlockSpec((1, FB, D_MODEL), lambda t, e, f: (e, f, 0)),
        ],
        out_specs=pl.BlockSpec((TB, D_MODEL), lambda t, e, f: (t, 0)),
        out_shape=jax.ShapeDtypeStruct((N_TOK, D_MODEL), jnp.float32),
        compiler_params=pltpu.CompilerParams(
            dimension_semantics=("parallel", "arbitrary", "arbitrary"),
        ),
    )(x_bf, gates, w1_bf, w2_bf)
    return out


# speedup vs baseline: 1.3494x; 1.2503x over previous
import functools

import jax
import jax.numpy as jnp
from jax import lax
from jax.experimental import pallas as pl
from jax.experimental.pallas import tpu as pltpu
from jax.experimental.pallas import tpu_sc as plsc

D_MODEL = 1024
D_FF = 4096
N_EXP = 8
N_TOK = 4096

BT = 256
NROWS = 2 * N_TOK + N_EXP * BT
NB = NROWS // BT
META_LEN = 80

NC, NS = 2, 16
NW = NC * NS
TPW = N_TOK // NW
SPW = NROWS // NW

_MESH = plsc.VectorSubcoreMesh(core_axis_name="c", subcore_axis_name="s",
                               num_cores=NC, num_subcores=NS)


def _wid():
    return lax.axis_index("s") * NC + lax.axis_index("c")


def _lane():
    return lax.iota(jnp.int32, 16)


def _splat(vec, i):
    return vec.at[jnp.full((16,), i, jnp.int32)].get(mode="promise_in_bounds")


def _router_body(x_ref, wg_ref, idx_ref, gat_ref):
    lg = lax.dot_general(wg_ref[...], x_ref[...],
                         (((0,), (1,)), ((), ())),
                         preferred_element_type=jnp.float32)
    m = jnp.max(lg, axis=0, keepdims=True)
    e = jnp.exp(lg - m)
    p = e / jnp.sum(e, axis=0, keepdims=True)
    rows = lax.broadcasted_iota(jnp.int32, p.shape, 0)
    v1 = jnp.max(p, axis=0, keepdims=True)
    i1 = jnp.min(jnp.where(p == v1, rows, N_EXP), axis=0, keepdims=True)
    p2 = jnp.where(rows == i1, -1.0, p)
    v2 = jnp.max(p2, axis=0, keepdims=True)
    i2 = jnp.min(jnp.where(p2 == v2, rows, N_EXP), axis=0, keepdims=True)
    den = v1 + v2 + 1e-9
    idx_ref[...] = jnp.concatenate([i1, i2], axis=0)
    gat_ref[...] = jnp.concatenate([v1 / den, v2 / den], axis=0)


def _router(x, w_gate):
    return pl.pallas_call(
        _router_body,
        in_specs=[pl.BlockSpec((N_TOK, D_MODEL), lambda: (0, 0)),
                  pl.BlockSpec((D_MODEL, N_EXP), lambda: (0, 0))],
        out_specs=[pl.BlockSpec((2, N_TOK), lambda: (0, 0)),
                   pl.BlockSpec((2, N_TOK), lambda: (0, 0))],
        out_shape=[jax.ShapeDtypeStruct((2, N_TOK), jnp.int32),
                   jax.ShapeDtypeStruct((2, N_TOK), jnp.float32)],
    )(x, w_gate)


@functools.partial(
    pl.kernel, mesh=_MESH,
    compiler_params=pltpu.CompilerParams(needs_layout_passes=False),
    out_type=jax.ShapeDtypeStruct((NW, 16), jnp.int32),
    scratch_types=[pltpu.VMEM((TPW,), jnp.int32),
                   pltpu.VMEM((16,), jnp.int32)],
)
def _hist_kernel(idx_hbm, hist_hbm, iv, cv):
    w = _wid()
    base = w * TPW
    lane = _lane()
    cnt = jnp.zeros((16,), jnp.int32)
    for k in range(2):
        pltpu.sync_copy(idx_hbm.at[k, pl.ds(base, TPW)], iv)
        for c in range(TPW // 16):
            ids = iv[pl.ds(16 * c, 16)]
            for ex in range(N_EXP):
                n = plsc.all_reduce_population_count(ids == ex)
                cnt = cnt + jnp.where(lane == ex, n, 0)
    cv[...] = cnt
    pltpu.sync_copy(cv, hist_hbm.at[w])


@functools.partial(
    pl.kernel, mesh=_MESH,
    compiler_params=pltpu.CompilerParams(needs_layout_passes=False),
    out_type=[jax.ShapeDtypeStruct((NROWS,), jnp.int32),
              jax.ShapeDtypeStruct((NROWS,), jnp.float32),
              jax.ShapeDtypeStruct((2, N_TOK), jnp.int32),
              jax.ShapeDtypeStruct((META_LEN,), jnp.int32)],
    scratch_types=[pltpu.VMEM((NW, 16), jnp.int32),
                   pltpu.VMEM((TPW,), jnp.int32),
                   pltpu.VMEM((TPW,), jnp.int32),
                   pltpu.VMEM((TPW,), jnp.float32),
                   pltpu.VMEM((TPW,), jnp.float32),
                   pltpu.VMEM((TPW,), jnp.int32),
                   pltpu.VMEM((TPW,), jnp.int32),
                   pltpu.VMEM((TPW,), jnp.int32),
                   pltpu.VMEM((16,), jnp.int32),
                   pltpu.SemaphoreType.DMA],
)
def _dispatch_kernel(idx_hbm, gat_hbm, hist_hbm,
                     tok_hbm, gd_hbm, slots_hbm, meta_hbm,
                     hv, i1v, i2v, g1v, g2v, d1v, d2v, tv, bev, sem):
    w = _wid()
    base = w * TPW
    lane = _lane()

    pltpu.sync_copy(idx_hbm.at[0, pl.ds(base, TPW)], i1v)
    pltpu.sync_copy(idx_hbm.at[1, pl.ds(base, TPW)], i2v)
    pltpu.sync_copy(gat_hbm.at[0, pl.ds(base, TPW)], g1v)
    pltpu.sync_copy(gat_hbm.at[1, pl.ds(base, TPW)], g2v)
    pltpu.sync_copy(hist_hbm, hv)

    tot = hv[0]
    for r in range(1, NW):
        tot = tot + hv[r]
    pad = ((tot + (BT - 1)) >> 8) << 8
    incl = plsc.cumsum(pad)
    excl = incl - pad
    padded_total = _splat(incl, N_EXP - 1)

    prior = jnp.zeros((16,), jnp.int32)
    for r in range(NW):
        take = jnp.where(r < w, 1, 0)
        prior = prior + take * hv[r]

    cursor = excl + prior
    cur = [_splat(cursor, ex) for ex in range(N_EXP)]

    for c in range(TPW // 16):
        tv[pl.ds(16 * c, 16)] = base + 16 * c + lane
    for iv, dv in ((i1v, d1v), (i2v, d2v)):
        for c in range(TPW // 16):
            ids = iv[pl.ds(16 * c, 16)]
            dst = jnp.zeros((16,), jnp.int32)
            for ex in range(N_EXP):
                m = ids == ex
                mi = jnp.where(m, 1, 0)
                rank = plsc.cumsum(mi) - 1
                dst = jnp.where(m, cur[ex] + rank, dst)
                cur[ex] = cur[ex] + plsc.all_reduce_population_count(m)
            dv[pl.ds(16 * c, 16)] = dst

    pltpu.sync_copy(d1v, slots_hbm.at[0, pl.ds(base, TPW)])
    pltpu.sync_copy(d2v, slots_hbm.at[1, pl.ds(base, TPW)])
    pltpu.async_copy(tv, tok_hbm.at[d1v], sem).wait()
    pltpu.async_copy(tv, tok_hbm.at[d2v], sem).wait()
    pltpu.async_copy(g1v, gd_hbm.at[d1v], sem).wait()
    pltpu.async_copy(g2v, gd_hbm.at[d2v], sem).wait()

    @pl.when(w == 0)
    def _meta():
        inc_s = [_splat(incl, ex) for ex in range(N_EXP)]
        last = padded_total - BT
        for c in range(4):
            bs = (lane + 16 * c) * BT
            bsc = jnp.minimum(bs, last)
            be = jnp.zeros((16,), jnp.int32)
            for ex in range(N_EXP):
                be = be + jnp.where(bsc >= inc_s[ex], 1, 0)
            bev[...] = be
            pltpu.sync_copy(bev, meta_hbm.at[pl.ds(16 * c, 16)])
        bev[...] = jnp.zeros((16,), jnp.int32) + (padded_total >> 8)
        pltpu.sync_copy(bev, meta_hbm.at[pl.ds(64, 16)])


@functools.partial(
    pl.kernel, mesh=_MESH,
    compiler_params=pltpu.CompilerParams(needs_layout_passes=False),
    out_type=jax.ShapeDtypeStruct((NROWS, D_MODEL), jnp.float32),
    scratch_types=[pltpu.VMEM((SPW,), jnp.int32),
                   pltpu.VMEM((64,), jnp.int32),
                   pltpu.VMEM((64, D_MODEL), jnp.float32),
                   pltpu.SemaphoreType.DMA],
)
def _gather_kernel(x_hbm, tok_hbm, xd_hbm, tid, idx64, buf, sem):
    w = _wid()
    s0 = w * SPW
    pltpu.sync_copy(tok_hbm.at[pl.ds(s0, SPW)], tid)
    for c in range(SPW // 16):
        v = tid[pl.ds(16 * c, 16)]
        tid[pl.ds(16 * c, 16)] = jnp.minimum(jnp.maximum(v, 0), N_TOK - 1)
    for s in range(SPW // 64):
        for c in range(4):
            idx64[pl.ds(16 * c, 16)] = tid[pl.ds(64 * s + 16 * c, 16)]
        pltpu.async_copy(x_hbm.at[idx64], buf, sem).wait()
        pltpu.sync_copy(buf, xd_hbm.at[pl.ds(s0 + 64 * s, 64)])


def _ffn_body(meta_ref, xd_ref, w1_ref, w2_ref, gd_ref, y_ref):
    b = pl.program_id(0)
    n_active = meta_ref[64]

    @pl.when(b < n_active)
    def _go():
        xb = xd_ref[...].astype(jnp.bfloat16)
        h = jnp.dot(xb, w1_ref[0], preferred_element_type=jnp.float32)
        h = jnp.maximum(h, 0.0).astype(jnp.bfloat16)
        y = jnp.dot(h, w2_ref[0], preferred_element_type=jnp.float32)
        g = gd_ref[0]
        rows = lax.broadcasted_iota(jnp.int32, (BT, BT), 0)
        cols = lax.broadcasted_iota(jnp.int32, (BT, BT), 1)
        dg = jnp.where(rows == cols, g, 0.0)
        y_ref[...] = jnp.dot(dg, y, preferred_element_type=jnp.float32)


def _ffn(meta, x_disp, w1_bf, w2_bf, gate_disp):
    grid_spec = pltpu.PrefetchScalarGridSpec(
        num_scalar_prefetch=1,
        grid=(NB,),
        in_specs=[
            pl.BlockSpec((BT, D_MODEL), lambda b, be: (b, 0)),
            pl.BlockSpec((1, D_MODEL, D_FF), lambda b, be: (be[b], 0, 0)),
            pl.BlockSpec((1, D_FF, D_MODEL), lambda b, be: (be[b], 0, 0)),
            pl.BlockSpec((1, 1, BT), lambda b, be: (b, 0, 0)),
        ],
        out_specs=pl.BlockSpec((BT, D_MODEL), lambda b, be: (b, 0)),
    )
    return pl.pallas_call(
        _ffn_body,
        grid_spec=grid_spec,
        out_shape=jax.ShapeDtypeStruct((NROWS, D_MODEL), jnp.float32),
        compiler_params=pltpu.CompilerParams(
            dimension_semantics=("arbitrary",),
        ),
    )(meta, x_disp, w1_bf, w2_bf, gate_disp.reshape(NB, 1, BT))


@functools.partial(
    pl.kernel, mesh=_MESH,
    compiler_params=pltpu.CompilerParams(needs_layout_passes=False),
    out_type=jax.ShapeDtypeStruct((N_TOK, D_MODEL), jnp.float32),
    scratch_types=[pltpu.VMEM((TPW,), jnp.int32),
                   pltpu.VMEM((TPW,), jnp.int32),
                   pltpu.VMEM((32,), jnp.int32),
                   pltpu.VMEM((32,), jnp.int32),
                   pltpu.VMEM((32, D_MODEL), jnp.float32),
                   pltpu.VMEM((32, D_MODEL), jnp.float32),
                   pltpu.VMEM((32, D_MODEL), jnp.float32),
                   pltpu.SemaphoreType.DMA],
)
def _combine_kernel(y_hbm, slots_hbm, out_hbm,
                    s1v, s2v, ia, ib, b1, b2, bo, sem):
    w = _wid()
    base = w * TPW
    pltpu.sync_copy(slots_hbm.at[0, pl.ds(base, TPW)], s1v)
    pltpu.sync_copy(slots_hbm.at[1, pl.ds(base, TPW)], s2v)
    for s in range(TPW // 32):
        for c in range(2):
            ia[pl.ds(16 * c, 16)] = s1v[pl.ds(32 * s + 16 * c, 16)]
            ib[pl.ds(16 * c, 16)] = s2v[pl.ds(32 * s + 16 * c, 16)]
        cp1 = pltpu.async_copy(y_hbm.at[ia], b1, sem)
        cp2 = pltpu.async_copy(y_hbm.at[ib], b2, sem)
        cp1.wait()
        cp2.wait()
        for r in range(32):
            def _add(j, _, r=r):
                bo[r, pl.ds(16 * j, 16)] = (b1[r, pl.ds(16 * j, 16)]
                                            + b2[r, pl.ds(16 * j, 16)])
                return 0

            lax.fori_loop(0, D_MODEL // 16, _add, 0)
        pltpu.sync_copy(bo, out_hbm.at[pl.ds(base + 32 * s, 32)])


@jax.jit
def kernel(x, w_gate, w1, w2):
    idx2, gates2 = _router(x, w_gate)
    hist = _hist_kernel(idx2)
    tok_sorted, gate_disp, slots, meta = _dispatch_kernel(idx2, gates2, hist)
    x_disp = _gather_kernel(x, tok_sorted)
    w1_bf = w1.astype(jnp.bfloat16)
    w2_bf = w2.astype(jnp.bfloat16)
    y_disp = _ffn(meta, x_disp, w1_bf, w2_bf, gate_disp)
    out = _combine_kernel(y_disp, slots)
    return out


# speedup vs baseline: 1.8674x; 1.3838x over previous
import functools

import jax
import jax.numpy as jnp
from jax import lax
from jax.experimental import pallas as pl
from jax.experimental.pallas import tpu as pltpu
from jax.experimental.pallas import tpu_sc as plsc

D_MODEL = 1024
D_FF = 4096
N_EXP = 8
N_TOK = 4096

BT = 256
NROWS = 2 * N_TOK + N_EXP * BT
NB = NROWS // BT
META_LEN = 80

NC, NS = 2, 16
NW = NC * NS
TPW = N_TOK // NW
SPW = NROWS // NW

_MESH = plsc.VectorSubcoreMesh(core_axis_name="c", subcore_axis_name="s",
                               num_cores=NC, num_subcores=NS)


def _wid():
    return lax.axis_index("s") * NC + lax.axis_index("c")


def _lane():
    return lax.iota(jnp.int32, 16)


def _splat(vec, i):
    return vec.at[jnp.full((16,), i, jnp.int32)].get(mode="promise_in_bounds")


def _router_body(x_ref, wg_ref, idx_ref, gat_ref):
    lg = lax.dot_general(wg_ref[...], x_ref[...],
                         (((0,), (1,)), ((), ())),
                         preferred_element_type=jnp.float32)
    m = jnp.max(lg, axis=0, keepdims=True)
    e = jnp.exp(lg - m)
    p = e / jnp.sum(e, axis=0, keepdims=True)
    rows = lax.broadcasted_iota(jnp.int32, p.shape, 0)
    v1 = jnp.max(p, axis=0, keepdims=True)
    i1 = jnp.min(jnp.where(p == v1, rows, N_EXP), axis=0, keepdims=True)
    p2 = jnp.where(rows == i1, -1.0, p)
    v2 = jnp.max(p2, axis=0, keepdims=True)
    i2 = jnp.min(jnp.where(p2 == v2, rows, N_EXP), axis=0, keepdims=True)
    den = v1 + v2 + 1e-9
    idx_ref[...] = jnp.concatenate([i1, i2], axis=0)
    gat_ref[...] = jnp.concatenate([v1 / den, v2 / den], axis=0)


def _router(x, w_gate):
    return pl.pallas_call(
        _router_body,
        in_specs=[pl.BlockSpec((N_TOK, D_MODEL), lambda: (0, 0)),
                  pl.BlockSpec((D_MODEL, N_EXP), lambda: (0, 0))],
        out_specs=[pl.BlockSpec((2, N_TOK), lambda: (0, 0)),
                   pl.BlockSpec((2, N_TOK), lambda: (0, 0))],
        out_shape=[jax.ShapeDtypeStruct((2, N_TOK), jnp.int32),
                   jax.ShapeDtypeStruct((2, N_TOK), jnp.float32)],
    )(x, w_gate)


@functools.partial(
    pl.kernel, mesh=_MESH,
    compiler_params=pltpu.CompilerParams(needs_layout_passes=False),
    out_type=jax.ShapeDtypeStruct((NW, 16), jnp.int32),
    scratch_types=[pltpu.VMEM((TPW,), jnp.int32),
                   pltpu.VMEM((16,), jnp.int32)],
)
def _hist_kernel(idx_hbm, hist_hbm, iv, cv):
    w = _wid()
    base = w * TPW
    lane = _lane()
    cnt = jnp.zeros((16,), jnp.int32)
    for k in range(2):
        pltpu.sync_copy(idx_hbm.at[k, pl.ds(base, TPW)], iv)
        for c in range(TPW // 16):
            ids = iv[pl.ds(16 * c, 16)]
            for ex in range(N_EXP):
                n = plsc.all_reduce_population_count(ids == ex)
                cnt = cnt + jnp.where(lane == ex, n, 0)
    cv[...] = cnt
    pltpu.sync_copy(cv, hist_hbm.at[w])


@functools.partial(
    pl.kernel, mesh=_MESH,
    compiler_params=pltpu.CompilerParams(needs_layout_passes=False),
    out_type=[jax.ShapeDtypeStruct((NROWS, D_MODEL), jnp.float32),
              jax.ShapeDtypeStruct((NROWS,), jnp.float32),
              jax.ShapeDtypeStruct((2, N_TOK), jnp.int32),
              jax.ShapeDtypeStruct((META_LEN,), jnp.int32)],
    scratch_types=[pltpu.VMEM((NW, 16), jnp.int32),
                   pltpu.VMEM((TPW,), jnp.int32),
                   pltpu.VMEM((TPW,), jnp.int32),
                   pltpu.VMEM((TPW,), jnp.float32),
                   pltpu.VMEM((TPW,), jnp.float32),
                   pltpu.VMEM((TPW,), jnp.int32),
                   pltpu.VMEM((TPW,), jnp.int32),
                   pltpu.VMEM((16,), jnp.int32),
                   [pltpu.VMEM((32, D_MODEL), jnp.float32) for _ in range(2)],
                   [pltpu.VMEM((32,), jnp.int32) for _ in range(4)],
                   pltpu.SemaphoreType.DMA],
)
def _dispatch_kernel(idx_hbm, gat_hbm, hist_hbm, x_hbm,
                     xd_hbm, gd_hbm, slots_hbm, meta_hbm,
                     hv, i1v, i2v, g1v, g2v, d1v, d2v, bev, xbufs, dcs, sem):
    w = _wid()
    base = w * TPW
    lane = _lane()

    pltpu.sync_copy(idx_hbm.at[0, pl.ds(base, TPW)], i1v)
    pltpu.sync_copy(idx_hbm.at[1, pl.ds(base, TPW)], i2v)
    pltpu.sync_copy(gat_hbm.at[0, pl.ds(base, TPW)], g1v)
    pltpu.sync_copy(gat_hbm.at[1, pl.ds(base, TPW)], g2v)
    pltpu.sync_copy(hist_hbm, hv)

    tot = hv[0]
    for r in range(1, NW):
        tot = tot + hv[r]
    pad = ((tot + (BT - 1)) >> 8) << 8
    incl = plsc.cumsum(pad)
    excl = incl - pad
    padded_total = _splat(incl, N_EXP - 1)

    prior = jnp.zeros((16,), jnp.int32)
    for r in range(NW):
        take = jnp.where(r < w, 1, 0)
        prior = prior + take * hv[r]

    cursor = excl + prior
    cur = [_splat(cursor, ex) for ex in range(N_EXP)]

    for iv, dv in ((i1v, d1v), (i2v, d2v)):
        for c in range(TPW // 16):
            ids = iv[pl.ds(16 * c, 16)]
            dst = jnp.zeros((16,), jnp.int32)
            for ex in range(N_EXP):
                m = ids == ex
                mi = jnp.where(m, 1, 0)
                rank = plsc.cumsum(mi) - 1
                dst = jnp.where(m, cur[ex] + rank, dst)
                cur[ex] = cur[ex] + plsc.all_reduce_population_count(m)
            dv[pl.ds(16 * c, 16)] = dst

    pltpu.sync_copy(d1v, slots_hbm.at[0, pl.ds(base, TPW)])
    pltpu.sync_copy(d2v, slots_hbm.at[1, pl.ds(base, TPW)])
    cg1 = pltpu.async_copy(g1v, gd_hbm.at[d1v], sem)
    cg2 = pltpu.async_copy(g2v, gd_hbm.at[d2v], sem)

    pending = [None, None]
    for s in range(TPW // 32):
        xb = xbufs[s % 2]
        c1, c2 = dcs[2 * (s % 2)], dcs[2 * (s % 2) + 1]
        if pending[s % 2] is not None:
            pending[s % 2][0].wait()
            pending[s % 2][1].wait()
        for c in range(2):
            c1[pl.ds(16 * c, 16)] = d1v[pl.ds(32 * s + 16 * c, 16)]
            c2[pl.ds(16 * c, 16)] = d2v[pl.ds(32 * s + 16 * c, 16)]
        pltpu.sync_copy(x_hbm.at[pl.ds(base + 32 * s, 32)], xb)
        pending[s % 2] = (pltpu.async_copy(xb, xd_hbm.at[c1], sem),
                          pltpu.async_copy(xb, xd_hbm.at[c2], sem))
    for pr in pending:
        pr[0].wait()
        pr[1].wait()
    cg1.wait()
    cg2.wait()

    @pl.when(w == 0)
    def _meta():
        inc_s = [_splat(incl, ex) for ex in range(N_EXP)]
        last = padded_total - BT
        for c in range(4):
            bs = (lane + 16 * c) * BT
            bsc = jnp.minimum(bs, last)
            be = jnp.zeros((16,), jnp.int32)
            for ex in range(N_EXP):
                be = be + jnp.where(bsc >= inc_s[ex], 1, 0)
            bev[...] = be
            pltpu.sync_copy(bev, meta_hbm.at[pl.ds(16 * c, 16)])
        bev[...] = jnp.zeros((16,), jnp.int32) + (padded_total >> 8)
        pltpu.sync_copy(bev, meta_hbm.at[pl.ds(64, 16)])


def _ffn_body(meta_ref, xd_ref, w1_ref, w2_ref, gd_ref, y_ref):
    b = pl.program_id(0)
    n_active = meta_ref[64]

    @pl.when(b < n_active)
    def _go():
        xb = xd_ref[...].astype(jnp.bfloat16)
        h = jnp.dot(xb, w1_ref[0], preferred_element_type=jnp.float32)
        h = jnp.maximum(h, 0.0).astype(jnp.bfloat16)
        y = jnp.dot(h, w2_ref[0], preferred_element_type=jnp.float32)
        g = gd_ref[0].reshape(BT, 1)
        y_ref[...] = y * g


def _ffn(meta, x_disp, w1_bf, w2_bf, gate_disp):
    grid_spec = pltpu.PrefetchScalarGridSpec(
        num_scalar_prefetch=1,
        grid=(NB,),
        in_specs=[
            pl.BlockSpec((BT, D_MODEL), lambda b, be: (b, 0)),
            pl.BlockSpec((1, D_MODEL, D_FF), lambda b, be: (be[b], 0, 0)),
            pl.BlockSpec((1, D_FF, D_MODEL), lambda b, be: (be[b], 0, 0)),
            pl.BlockSpec((1, 1, BT), lambda b, be: (b, 0, 0)),
        ],
        out_specs=pl.BlockSpec((BT, D_MODEL), lambda b, be: (b, 0)),
    )
    return pl.pallas_call(
        _ffn_body,
        grid_spec=grid_spec,
        out_shape=jax.ShapeDtypeStruct((NROWS, D_MODEL), jnp.float32),
        compiler_params=pltpu.CompilerParams(
            dimension_semantics=("arbitrary",),
        ),
    )(meta, x_disp, w1_bf, w2_bf, gate_disp.reshape(NB, 1, BT))


@functools.partial(
    pl.kernel, mesh=_MESH,
    compiler_params=pltpu.CompilerParams(needs_layout_passes=False),
    out_type=jax.ShapeDtypeStruct((N_TOK, D_MODEL), jnp.float32),
    scratch_types=[pltpu.VMEM((TPW,), jnp.int32),
                   pltpu.VMEM((TPW,), jnp.int32),
                   pltpu.VMEM((32,), jnp.int32),
                   pltpu.VMEM((32,), jnp.int32),
                   pltpu.VMEM((32, D_MODEL), jnp.float32),
                   pltpu.VMEM((32, D_MODEL), jnp.float32),
                   pltpu.VMEM((32, D_MODEL), jnp.float32),
                   pltpu.SemaphoreType.DMA],
)
def _combine_kernel(y_hbm, slots_hbm, out_hbm,
                    s1v, s2v, ia, ib, b1, b2, bo, sem):
    w = _wid()
    base = w * TPW
    pltpu.sync_copy(slots_hbm.at[0, pl.ds(base, TPW)], s1v)
    pltpu.sync_copy(slots_hbm.at[1, pl.ds(base, TPW)], s2v)
    for s in range(TPW // 32):
        for c in range(2):
            ia[pl.ds(16 * c, 16)] = s1v[pl.ds(32 * s + 16 * c, 16)]
            ib[pl.ds(16 * c, 16)] = s2v[pl.ds(32 * s + 16 * c, 16)]
        cp1 = pltpu.async_copy(y_hbm.at[ia], b1, sem)
        cp2 = pltpu.async_copy(y_hbm.at[ib], b2, sem)
        cp1.wait()
        cp2.wait()
        for r in range(32):
            def _add(j, _, r=r):
                bo[r, pl.ds(16 * j, 16)] = (b1[r, pl.ds(16 * j, 16)]
                                            + b2[r, pl.ds(16 * j, 16)])
                return 0

            lax.fori_loop(0, D_MODEL // 16, _add, 0)
        pltpu.sync_copy(bo, out_hbm.at[pl.ds(base + 32 * s, 32)])


@jax.jit
def kernel(x, w_gate, w1, w2):
    idx2, gates2 = _router(x, w_gate)
    hist = _hist_kernel(idx2)
    x_disp, gate_disp, slots, meta = _dispatch_kernel(idx2, gates2, hist, x)
    w1_bf = w1.astype(jnp.bfloat16)
    w2_bf = w2.astype(jnp.bfloat16)
    y_disp = _ffn(meta, x_disp, w1_bf, w2_bf, gate_disp)
    out = _combine_kernel(y_disp, slots)
    return out
